# baseline (speedup 1.0000x reference)
import jax
import jax.numpy as jnp
from jax.experimental import pallas as pl
from jax.experimental.pallas import tpu as pltpu

EMBED_DIM = 768
HIDDEN_DIM = 768
NUM_EXPERTS = 8
TOP_K = 2


def _moe_kernel(x_ref, wg_ref, w1_ref, b1_ref, w2_ref, b2_ref,
                out_ref, aux_ref, xb16_ref, wcomb_ref):
    e = pl.program_id(0)

    @pl.when(e == 0)
    def _router():
        x = x_ref[...]
        logits = jax.lax.dot_general(
            x, wg_ref[...], (((1,), (1,)), ((), ())),
            preferred_element_type=jnp.float32,
        )
        m = jnp.max(logits, axis=-1, keepdims=True)
        ex = jnp.exp(logits - m)
        probs = ex / jnp.sum(ex, axis=-1, keepdims=True)

        T, E = probs.shape
        idx = jax.lax.broadcasted_iota(jnp.int32, (T, E), 1)
        big = jnp.int32(E)
        m1 = jnp.max(probs, axis=-1, keepdims=True)
        i1 = jnp.min(jnp.where(probs == m1, idx, big), axis=-1, keepdims=True)
        masked = jnp.where(idx == i1, -jnp.inf, probs)
        m2 = jnp.max(masked, axis=-1, keepdims=True)
        i2 = jnp.min(jnp.where(masked == m2, idx, big), axis=-1, keepdims=True)

        denom = m1 + m2
        onehot1 = (idx == i1).astype(jnp.float32)
        onehot2 = (idx == i2).astype(jnp.float32)
        wcomb = (m1 / denom) * onehot1 + (m2 / denom) * onehot2
        wcomb_ref[...] = jnp.transpose(wcomb)

        f = jnp.sum(onehot1 + onehot2, axis=0) / jnp.float32(T)
        p = jnp.sum(probs, axis=0) / jnp.float32(T)
        aux_ref[...] = (jnp.float32(NUM_EXPERTS) * jnp.sum(f * p)).reshape(1, 1)

        xb16_ref[...] = x.astype(jnp.bfloat16)

    xb = xb16_ref[...]
    w1 = w1_ref[0].astype(jnp.bfloat16)
    h = jax.lax.dot_general(
        xb, w1, (((1,), (1,)), ((), ())), preferred_element_type=jnp.float32
    ) + b1_ref[0]
    h = h * jax.nn.sigmoid(h)
    w2 = w2_ref[0].astype(jnp.bfloat16)
    eo = jax.lax.dot_general(
        h.astype(jnp.bfloat16), w2, (((1,), (1,)), ((), ())),
        preferred_element_type=jnp.float32,
    ) + b2_ref[0]
    wcol = jnp.transpose(wcomb_ref[pl.ds(e, 1), :])

    @pl.when(e == 0)
    def _first():
        out_ref[...] = wcol * eo

    @pl.when(e > 0)
    def _acc():
        out_ref[...] += wcol * eo


def kernel(x, Wg, W1, b1, W2, b2):
    Bq, Sq, D = x.shape
    T = Bq * Sq
    E = NUM_EXPERTS
    H = HIDDEN_DIM
    xf = x.reshape(T, D)

    out, aux = pl.pallas_call(
        _moe_kernel,
        grid=(E,),
        in_specs=[
            pl.BlockSpec((T, D), lambda e: (0, 0)),
            pl.BlockSpec((E, D), lambda e: (0, 0)),
            pl.BlockSpec((1, H, D), lambda e: (e, 0, 0)),
            pl.BlockSpec((1, 1, H), lambda e: (e, 0, 0)),
            pl.BlockSpec((1, D, H), lambda e: (e, 0, 0)),
            pl.BlockSpec((1, 1, D), lambda e: (e, 0, 0)),
        ],
        out_specs=(
            pl.BlockSpec((T, D), lambda e: (0, 0)),
            pl.BlockSpec((1, 1), lambda e: (0, 0)),
        ),
        out_shape=(
            jax.ShapeDtypeStruct((T, D), jnp.float32),
            jax.ShapeDtypeStruct((1, 1), jnp.float32),
        ),
        scratch_shapes=[
            pltpu.VMEM((T, D), jnp.bfloat16),
            pltpu.VMEM((E, T), jnp.float32),
        ],
    )(xf, Wg, W1, b1.reshape(E, 1, H), W2, b2.reshape(E, 1, D))

    return out.reshape(Bq, Sq, D), aux.reshape(())
